# baseline (speedup 1.0000x reference)
import functools

import jax
import jax.numpy as jnp
from jax.experimental import pallas as pl
from jax.experimental.pallas import tpu as pltpu
from jax.experimental.pallas import tpu_sc as plsc

L = 128
P = 4
U = 32
D = L * P
TB = 2048


@functools.partial(
    pl.kernel,
    out_type=jax.ShapeDtypeStruct((U, D), jnp.float32),
    mesh=plsc.VectorSubcoreMesh(core_axis_name="c", subcore_axis_name="s"),
    compiler_params=pltpu.CompilerParams(needs_layout_passes=False),
    scratch_types=[
        pltpu.VMEM((U,), jnp.int32),
        pltpu.VMEM((U,), jnp.int32),
        pltpu.VMEM((D,), jnp.float32),
    ],
)
def _build_g(ui_hbm, mui_hbm, g_hbm, uiv, muiv, row_v):
    nc = plsc.get_sparse_core_info().num_cores
    wid = jax.lax.axis_index("s") * nc + jax.lax.axis_index("c")
    pltpu.sync_copy(ui_hbm, uiv)
    pltpu.sync_copy(mui_hbm, muiv)
    lane = jax.lax.iota(jnp.int32, 16)
    hi = jnp.full((16,), wid // 16, jnp.int32) == 1
    u_vec = jnp.where(hi, uiv[pl.ds(16, 16)], uiv[pl.ds(0, 16)])
    m_vec = jnp.where(hi, muiv[pl.ds(16, 16)], muiv[pl.ds(0, 16)])
    mask = lane == jnp.full((16,), wid % 16, jnp.int32)
    zero16 = jnp.zeros((16,), jnp.int32)
    uiw = jnp.full((16,), jnp.sum(jnp.where(mask, u_vec, zero16)), jnp.int32)
    muiw = jnp.full((16,), jnp.sum(jnp.where(mask, m_vec, zero16)), jnp.int32)
    for c in range(D // 16):
        l = (lane + c * 16) // P
        row_v[pl.ds(c * 16, 16)] = ((l == uiw).astype(jnp.float32)
                                    + (l == muiw).astype(jnp.float32))
    pltpu.sync_copy(row_v, g_hbm.at[wid])


def _head(x_ref, w1_ref, b1_ref, w2_ref, b2_ref, y_ref):
    bf = lambda a: a.astype(jnp.bfloat16)
    h = jnp.tanh(jnp.dot(bf(x_ref[...]), bf(w1_ref[...]),
                         preferred_element_type=jnp.float32) + b1_ref[...])
    logits = jnp.dot(bf(h), bf(w2_ref[...]),
                     preferred_element_type=jnp.float32) + b2_ref[...]
    m = jnp.max(logits, axis=1, keepdims=True)
    e = jnp.exp(logits - m)
    y_ref[...] = e / jnp.sum(e, axis=1, keepdims=True)


def _tail(x_ref, y_ref, g_ref, wd1_ref, bd1_ref, wd2_ref, bd2_ref, q_ref):
    x = x_ref[...]
    y = y_ref[...]
    bf = lambda a: a.astype(jnp.bfloat16)
    turn = x[:, P:2 * P]
    budgets = jnp.max(x[:, 0:P], axis=1, keepdims=True)
    pid = jnp.argmax(turn, axis=1)[:, None]

    j = jax.lax.broadcasted_iota(jnp.int32, (TB, D), 1)
    z = jnp.where(j == pid, 0.0, x)

    s_exp = jnp.dot(bf(y), bf(g_ref[...]),
                    preferred_element_type=jnp.float32)
    pcol = jax.lax.broadcasted_iota(jnp.int32, (P, D), 1) % P
    prow = jax.lax.broadcasted_iota(jnp.int32, (P, D), 0)
    f = (pcol == prow).astype(jnp.float32)
    tt_exp = jnp.dot(bf(budgets * turn), bf(f),
                     preferred_element_type=jnp.float32)
    x2 = z + s_exp * tt_exp

    hd = jnp.tanh(jnp.dot(bf(x2), bf(wd1_ref[...]),
                          preferred_element_type=jnp.float32) + bd1_ref[...])
    out = jnp.dot(bf(hd), bf(wd2_ref[...]),
                  preferred_element_type=jnp.float32) + bd2_ref[...]
    q_ref[...] = jnp.sum(out * turn, axis=1, keepdims=True)


def kernel(data, unit_indices, moveable_unit_indices,
           W1, b1, W2, b2, Wd1, bd1, Wd2, bd2):
    batch = data.shape[0]
    H = W1.shape[1]
    rep = lambda shape: pl.BlockSpec(shape, lambda i: (0, 0))
    g = _build_g(unit_indices.astype(jnp.int32),
                 moveable_unit_indices.astype(jnp.int32))
    y = pl.pallas_call(
        _head,
        grid=(batch // TB,),
        in_specs=[
            pl.BlockSpec((TB, D), lambda i: (i, 0)),
            rep((D, H)), rep((1, H)), rep((H, U)), rep((1, U)),
        ],
        out_specs=pl.BlockSpec((TB, U), lambda i: (i, 0)),
        out_shape=jax.ShapeDtypeStruct((batch, U), jnp.float32),
    )(data, W1, b1.reshape(1, -1), W2, b2.reshape(1, -1))
    q = pl.pallas_call(
        _tail,
        grid=(batch // TB,),
        in_specs=[
            pl.BlockSpec((TB, D), lambda i: (i, 0)),
            pl.BlockSpec((TB, U), lambda i: (i, 0)),
            rep((U, D)),
            rep((D, H)), rep((1, H)),
            rep((H, P)), rep((1, P)),
        ],
        out_specs=pl.BlockSpec((TB, 1), lambda i: (i, 0)),
        out_shape=jax.ShapeDtypeStruct((batch, 1), jnp.float32),
    )(data, y, g, Wd1, bd1.reshape(1, -1), Wd2, bd2.reshape(1, -1))
    return q.reshape(batch)


# speedup vs baseline: 1.2881x; 1.2881x over previous
import functools

import jax
import jax.numpy as jnp
from jax.experimental import pallas as pl
from jax.experimental.pallas import tpu as pltpu
from jax.experimental.pallas import tpu_sc as plsc

L = 128
P = 4
U = 32
D = L * P
TB = 2048


@functools.partial(
    pl.kernel,
    out_type=jax.ShapeDtypeStruct((U, D), jnp.float32),
    mesh=plsc.VectorSubcoreMesh(core_axis_name="c", subcore_axis_name="s"),
    compiler_params=pltpu.CompilerParams(needs_layout_passes=False),
    scratch_types=[
        pltpu.VMEM((2 * U,), jnp.int32),
        pltpu.VMEM((D,), jnp.float32),
    ],
)
def _build_g(idx_hbm, g_hbm, idxv, row_v):
    nc = plsc.get_sparse_core_info().num_cores
    wid = jax.lax.axis_index("s") * nc + jax.lax.axis_index("c")
    pltpu.sync_copy(idx_hbm, idxv)
    lane = jax.lax.iota(jnp.int32, 16)
    hi = jnp.full((16,), wid // 16, jnp.int32) == 1
    u_vec = jnp.where(hi, idxv[pl.ds(16, 16)], idxv[pl.ds(0, 16)])
    m_vec = jnp.where(hi, idxv[pl.ds(48, 16)], idxv[pl.ds(32, 16)])
    mask = lane == jnp.full((16,), wid % 16, jnp.int32)
    zero16 = jnp.zeros((16,), jnp.int32)
    uiw = jnp.full((16,), jnp.sum(jnp.where(mask, u_vec, zero16)), jnp.int32)
    muiw = jnp.full((16,), jnp.sum(jnp.where(mask, m_vec, zero16)), jnp.int32)
    for c in range(D // 16):
        l = (lane + c * 16) // P
        row_v[pl.ds(c * 16, 16)] = ((l == uiw).astype(jnp.float32)
                                    + (l == muiw).astype(jnp.float32))
    pltpu.sync_copy(row_v, g_hbm.at[wid])


def _fused(x_ref, g_ref, w1_ref, b1_ref, w2_ref, b2_ref,
           wd1_ref, bd1_ref, wd2_ref, bd2_ref, q_ref):
    x = x_ref[...]
    bf = lambda a: a.astype(jnp.bfloat16)
    h = jnp.tanh(jnp.dot(bf(x), bf(w1_ref[...]),
                         preferred_element_type=jnp.float32) + b1_ref[...])
    logits = jnp.dot(bf(h), bf(w2_ref[...]),
                     preferred_element_type=jnp.float32) + b2_ref[...]
    m = jnp.max(logits, axis=1, keepdims=True)
    e = jnp.exp(logits - m)
    y = e / jnp.sum(e, axis=1, keepdims=True)

    turn = x[:, P:2 * P]
    budgets = jnp.max(x[:, 0:P], axis=1, keepdims=True)
    pid = jnp.argmax(turn, axis=1)[:, None]

    j = jax.lax.broadcasted_iota(jnp.int32, (TB, D), 1)
    z = jnp.where(j == pid, 0.0, x)

    s_exp = jnp.dot(bf(y), bf(g_ref[...]),
                    preferred_element_type=jnp.float32)
    pcol = jax.lax.broadcasted_iota(jnp.int32, (P, D), 1) % P
    prow = jax.lax.broadcasted_iota(jnp.int32, (P, D), 0)
    f = (pcol == prow).astype(jnp.float32)
    tt_exp = jnp.dot(bf(budgets * turn), bf(f),
                     preferred_element_type=jnp.float32)
    x2 = z + s_exp * tt_exp

    hd = jnp.tanh(jnp.dot(bf(x2), bf(wd1_ref[...]),
                          preferred_element_type=jnp.float32) + bd1_ref[...])
    out = jnp.dot(bf(hd), bf(wd2_ref[...]),
                  preferred_element_type=jnp.float32) + bd2_ref[...]
    q_ref[...] = jnp.sum(out * turn, axis=1, keepdims=True)


def kernel(data, unit_indices, moveable_unit_indices,
           W1, b1, W2, b2, Wd1, bd1, Wd2, bd2):
    batch = data.shape[0]
    idx = jnp.concatenate([unit_indices.astype(jnp.int32),
                           moveable_unit_indices.astype(jnp.int32)])
    g = _build_g(idx)
    rep = lambda shape: pl.BlockSpec(shape, lambda i: (0, 0))
    q = pl.pallas_call(
        _fused,
        grid=(batch // TB,),
        in_specs=[
            pl.BlockSpec((TB, D), lambda i: (i, 0)),
            rep((U, D)),
            rep((D, H := W1.shape[1])), rep((1, H)),
            rep((H, U)), rep((1, U)),
            rep((D, H)), rep((1, H)),
            rep((H, P)), rep((1, P)),
        ],
        out_specs=pl.BlockSpec((TB, 1), lambda i: (i, 0)),
        out_shape=jax.ShapeDtypeStruct((batch, 1), jnp.float32),
    )(data, g, W1, b1.reshape(1, -1), W2, b2.reshape(1, -1),
      Wd1, bd1.reshape(1, -1), Wd2, bd2.reshape(1, -1))
    return q.reshape(batch)


# speedup vs baseline: 1.2924x; 1.0034x over previous
import functools

import jax
import jax.numpy as jnp
from jax.experimental import pallas as pl
from jax.experimental.pallas import tpu as pltpu
from jax.experimental.pallas import tpu_sc as plsc

L = 128
P = 4
U = 32
D = L * P
TB = 2048


@functools.partial(
    pl.kernel,
    out_type=jax.ShapeDtypeStruct((U, D), jnp.float32),
    mesh=plsc.VectorSubcoreMesh(core_axis_name="c", subcore_axis_name="s"),
    compiler_params=pltpu.CompilerParams(needs_layout_passes=False),
    scratch_types=[
        pltpu.VMEM((2 * U,), jnp.int32),
        pltpu.VMEM((D,), jnp.float32),
    ],
)
def _build_g(idx_hbm, g_hbm, idxv, row_v):
    nc = plsc.get_sparse_core_info().num_cores
    wid = jax.lax.axis_index("s") * nc + jax.lax.axis_index("c")
    pltpu.sync_copy(idx_hbm, idxv)
    lane = jax.lax.iota(jnp.int32, 16)
    hi = jnp.full((16,), wid // 16, jnp.int32) == 1
    u_vec = jnp.where(hi, idxv[pl.ds(16, 16)], idxv[pl.ds(0, 16)])
    m_vec = jnp.where(hi, idxv[pl.ds(48, 16)], idxv[pl.ds(32, 16)])
    mask = lane == jnp.full((16,), wid % 16, jnp.int32)
    zero16 = jnp.zeros((16,), jnp.int32)
    uiw = jnp.full((16,), jnp.sum(jnp.where(mask, u_vec, zero16)), jnp.int32)
    muiw = jnp.full((16,), jnp.sum(jnp.where(mask, m_vec, zero16)), jnp.int32)
    for c in range(D // 16):
        l = (lane + c * 16) // P
        row_v[pl.ds(c * 16, 16)] = ((l == uiw).astype(jnp.float32)
                                    + (l == muiw).astype(jnp.float32))
    pltpu.sync_copy(row_v, g_hbm.at[wid])


def _fused(x_ref, g_ref, w1_ref, b1_ref, w2_ref, b2_ref,
           wd1_ref, bd1_ref, wd2_ref, bd2_ref, q_ref):
    x = x_ref[...]
    bf = lambda a: a.astype(jnp.bfloat16)
    x16 = bf(x)
    h = jnp.tanh(bf(jnp.dot(x16, bf(w1_ref[...]),
                            preferred_element_type=jnp.float32)
                    + b1_ref[...]))
    logits = jnp.dot(h, bf(w2_ref[...]),
                     preferred_element_type=jnp.float32) + b2_ref[...]
    m = jnp.max(logits, axis=1, keepdims=True)
    e = jnp.exp(logits - m)
    y = e / jnp.sum(e, axis=1, keepdims=True)

    turn = x[:, P:2 * P]
    budgets = jnp.max(x[:, 0:P], axis=1, keepdims=True)
    pid = jnp.argmax(turn, axis=1)[:, None]

    j = jax.lax.broadcasted_iota(jnp.int32, (TB, D), 1)
    z = jnp.where(j == pid, jnp.bfloat16(0.0), x16)

    s_exp = bf(jnp.dot(bf(y), bf(g_ref[...]),
                       preferred_element_type=jnp.float32))
    pcol = jax.lax.broadcasted_iota(jnp.int32, (P, D), 1) % P
    prow = jax.lax.broadcasted_iota(jnp.int32, (P, D), 0)
    f = (pcol == prow).astype(jnp.bfloat16)
    tt_exp = bf(jnp.dot(bf(budgets * turn), f,
                        preferred_element_type=jnp.float32))
    x2 = z + s_exp * tt_exp

    hd = jnp.tanh(bf(jnp.dot(x2, bf(wd1_ref[...]),
                             preferred_element_type=jnp.float32)
                     + bd1_ref[...]))
    out = jnp.dot(hd, bf(wd2_ref[...]),
                  preferred_element_type=jnp.float32) + bd2_ref[...]
    q_ref[...] = jnp.sum(out * turn, axis=1, keepdims=True)


def kernel(data, unit_indices, moveable_unit_indices,
           W1, b1, W2, b2, Wd1, bd1, Wd2, bd2):
    batch = data.shape[0]
    idx = jnp.concatenate([unit_indices.astype(jnp.int32),
                           moveable_unit_indices.astype(jnp.int32)])
    g = _build_g(idx)
    rep = lambda shape: pl.BlockSpec(shape, lambda i: (0, 0))
    q = pl.pallas_call(
        _fused,
        grid=(batch // TB,),
        in_specs=[
            pl.BlockSpec((TB, D), lambda i: (i, 0)),
            rep((U, D)),
            rep((D, H := W1.shape[1])), rep((1, H)),
            rep((H, U)), rep((1, U)),
            rep((D, H)), rep((1, H)),
            rep((H, P)), rep((1, P)),
        ],
        out_specs=pl.BlockSpec((TB, 1), lambda i: (i, 0)),
        out_shape=jax.ShapeDtypeStruct((batch, 1), jnp.float32),
    )(data, g, W1, b1.reshape(1, -1), W2, b2.reshape(1, -1),
      Wd1, bd1.reshape(1, -1), Wd2, bd2.reshape(1, -1))
    return q.reshape(batch)
